# baseline (speedup 1.0000x reference)
import functools

import jax
import jax.numpy as jnp
from jax import lax
from jax.experimental import pallas as pl
from jax.experimental.pallas import tpu as pltpu
from jax.experimental.pallas import tpu_sc as plsc

_N = 50000
_D = 256
_E = 49999
_EP = 50176
_TILE_E = _EP // 32
_NGRP = _TILE_E // 16
_NPAD = 50176
_TRASH = 50100


def _gatherh_body(a_hbm, b_hbm, src_hbm, dst_hbm, ha_hbm, hb_hbm,
                  src_s, dst_s, a16a, b16a, a16b, b16b,
                  sema, semb, sema2, semb2):
    c = lax.axis_index("c")
    s = lax.axis_index("s")
    w = s * 2 + c
    ebase = w * _TILE_E

    pltpu.sync_copy(src_hbm.at[pl.ds(ebase, _TILE_E)], src_s)
    pltpu.sync_copy(dst_hbm.at[pl.ds(ebase, _TILE_E)], dst_s)

    def _issue(g, abuf, bbuf, s1, s2):
        idxs = src_s[pl.ds(g * 16, 16)]
        idxg = dst_s[pl.ds(g * 16, 16)]
        cpa = pltpu.async_copy(a_hbm.at[idxs], abuf, s1)
        cpb = pltpu.async_copy(b_hbm.at[idxg], bbuf, s2)
        return cpa, cpb

    def _store(g, abuf, bbuf):
        pltpu.sync_copy(abuf, ha_hbm.at[pl.ds(ebase + g * 16, 16)])
        pltpu.sync_copy(bbuf, hb_hbm.at[pl.ds(ebase + g * 16, 16)])

    _issue(0, a16a, b16a, sema, semb)

    def pair_body(it, _):
        g0 = 2 * it
        pltpu.make_async_copy(a_hbm.at[pl.ds(0, 16)], a16a, sema).wait()
        pltpu.make_async_copy(b_hbm.at[pl.ds(0, 16)], b16a, semb).wait()
        _issue(g0 + 1, a16b, b16b, sema2, semb2)
        _store(g0, a16a, b16a)
        pltpu.make_async_copy(a_hbm.at[pl.ds(0, 16)], a16b, sema2).wait()
        pltpu.make_async_copy(b_hbm.at[pl.ds(0, 16)], b16b, semb2).wait()

        @pl.when(it + 1 < _NGRP // 2)
        def _():
            _issue(g0 + 2, a16a, b16a, sema, semb)

        _store(g0 + 1, a16b, b16b)
        return 0

    lax.fori_loop(0, _NGRP // 2, pair_body, 0)


_gatherh_call = pl.kernel(
    _gatherh_body,
    out_type=[
        jax.ShapeDtypeStruct((_EP, _D), jnp.float32),
        jax.ShapeDtypeStruct((_EP, _D), jnp.float32),
    ],
    mesh=plsc.VectorSubcoreMesh(
        core_axis_name="c", subcore_axis_name="s", num_cores=2, num_subcores=16
    ),
    compiler_params=pltpu.CompilerParams(needs_layout_passes=False),
    scratch_types=[
        pltpu.VMEM((_TILE_E,), jnp.int32),
        pltpu.VMEM((_TILE_E,), jnp.int32),
        pltpu.VMEM((16, _D), jnp.float32),
        pltpu.VMEM((16, _D), jnp.float32),
        pltpu.VMEM((16, _D), jnp.float32),
        pltpu.VMEM((16, _D), jnp.float32),
        pltpu.SemaphoreType.DMA,
        pltpu.SemaphoreType.DMA,
        pltpu.SemaphoreType.DMA,
        pltpu.SemaphoreType.DMA,
    ],
)


def _count_body(dst_hbm, z_hbm, cnt_hbm, dst_s, cl):
    c = lax.axis_index("c")
    s = lax.axis_index("s")
    w = s * 2 + c
    ebase = w * _TILE_E

    pltpu.sync_copy(dst_hbm.at[pl.ds(ebase, _TILE_E)], dst_s)
    pltpu.sync_copy(z_hbm, cl)
    lanes = lax.iota(jnp.int32, 16)

    def _lane_take(v, idx):
        return lax.gather(
            v,
            idx[:, None],
            lax.GatherDimensionNumbers(
                offset_dims=(),
                collapsed_slice_dims=(0,),
                start_index_map=(0,),
            ),
            (1,),
            mode=lax.GatherScatterMode.PROMISE_IN_BOUNDS,
        )

    def chunk_body(i, _):
        d16 = dst_s[pl.ds(i * 16, 16)]
        later = jnp.zeros((16,), jnp.int32)
        earlier = jnp.zeros((16,), jnp.int32)
        for sh in range(1, 16):
            up = _lane_take(d16, jnp.minimum(lanes + sh, 15))
            later = later + jnp.where(
                (lanes + sh <= 15) & (d16 == up), 1, 0
            )
            dn = _lane_take(d16, jnp.maximum(lanes - sh, 0))
            earlier = earlier + jnp.where(
                (lanes - sh >= 0) & (d16 == dn), 1, 0
            )
        rep = earlier == 0
        total = (later + 1).astype(jnp.float32)
        plsc.addupdate_scatter(cl, [d16], total, mask=rep)
        return 0

    lax.fori_loop(0, _NGRP, chunk_body, 0)
    pltpu.sync_copy(cl, cnt_hbm.at[pl.ds(w * _NPAD, _NPAD)])


_count_call = pl.kernel(
    _count_body,
    out_type=jax.ShapeDtypeStruct((32 * _NPAD,), jnp.float32),
    mesh=plsc.VectorSubcoreMesh(
        core_axis_name="c", subcore_axis_name="s", num_cores=2, num_subcores=16
    ),
    compiler_params=pltpu.CompilerParams(needs_layout_passes=False),
    scratch_types=[
        pltpu.VMEM((_TILE_E,), jnp.int32),
        pltpu.VMEM((_NPAD,), jnp.float32),
    ],
)


def _scatter_kernel(ha_ref, hb_ref, bl_ref, w1c_ref, dst_ref, o_ref, h_s):
    @pl.when(pl.program_id(0) == 0)
    def _():
        o_ref[...] = jnp.zeros((_NPAD, _D), jnp.float32)

    h_s[...] = jnp.maximum(
        ha_ref[...] + hb_ref[...] + bl_ref[...] * w1c_ref[...], 0.0
    )

    def eb(j8, _):
        for u in range(16):
            j = j8 * 16 + u
            d = dst_ref[0, 0, j]
            o_ref[pl.ds(d, 1), :] += h_s[pl.ds(j, 1), :]
        return 0

    lax.fori_loop(0, _TILE_E // 16, eb, 0)


def _scatter_call(ha, hb, bl2, w1c, dst3d):
    return pl.pallas_call(
        _scatter_kernel,
        grid=(_EP // _TILE_E,),
        in_specs=[
            pl.BlockSpec((_TILE_E, _D), lambda i: (i, 0)),
            pl.BlockSpec((_TILE_E, _D), lambda i: (i, 0)),
            pl.BlockSpec((_TILE_E, 1), lambda i: (i, 0)),
            pl.BlockSpec((1, _D), lambda i: (0, 0)),
            pl.BlockSpec((1, 1, _TILE_E), lambda i: (i, 0, 0),
                         memory_space=pltpu.SMEM),
        ],
        out_specs=pl.BlockSpec((_NPAD, _D), lambda i: (0, 0)),
        out_shape=jax.ShapeDtypeStruct((_NPAD, _D), jnp.float32),
        scratch_shapes=[pltpu.VMEM((_TILE_E, _D), jnp.float32)],
        compiler_params=pltpu.CompilerParams(
            vmem_limit_bytes=60 * 1024 * 1024,
        ),
    )(ha, hb, bl2, w1c[None, :], dst3d)


def _reduce_kernel(c_ref, o_ref):
    o_ref[...] = jnp.sum(c_ref[...], axis=0, keepdims=True)


def _reduce_counts(cnt32, bn=6272):
    return pl.pallas_call(
        _reduce_kernel,
        grid=(_NPAD // bn,),
        in_specs=[pl.BlockSpec((32, bn), lambda i: (0, i))],
        out_specs=pl.BlockSpec((1, bn), lambda i: (0, i)),
        out_shape=jax.ShapeDtypeStruct((1, _NPAD), jnp.float32),
    )(cnt32)


def _mm2_kernel(x_ref, wa_ref, wb_ref, ba_ref, oa_ref, ob_ref):
    oa_ref[...] = (
        jnp.dot(x_ref[...], wa_ref[...], preferred_element_type=jnp.float32)
        + ba_ref[...]
    )
    ob_ref[...] = jnp.dot(
        x_ref[...], wb_ref[...], preferred_element_type=jnp.float32
    )


def _matmul_ab(x, wa_t, wb_t, ba, bm=400):
    M, K = x.shape
    N = wa_t.shape[1]
    return pl.pallas_call(
        _mm2_kernel,
        grid=(M // bm,),
        in_specs=[
            pl.BlockSpec((bm, K), lambda i: (i, 0)),
            pl.BlockSpec((K, N), lambda i: (0, 0)),
            pl.BlockSpec((K, N), lambda i: (0, 0)),
            pl.BlockSpec((1, N), lambda i: (0, 0)),
        ],
        out_specs=[
            pl.BlockSpec((bm, N), lambda i: (i, 0)),
            pl.BlockSpec((bm, N), lambda i: (i, 0)),
        ],
        out_shape=[
            jax.ShapeDtypeStruct((M, N), jnp.float32),
            jax.ShapeDtypeStruct((M, N), jnp.float32),
        ],
    )(x, wa_t, wb_t, ba[None, :])


def _post_ab_kernel(s_ref, w2_ref, x_ref, cnt_ref, b2_ref, wa_ref, wb_ref,
                    ba_ref, ox_ref, oa_ref, ob_ref):
    xn = (
        x_ref[...]
        + jnp.dot(s_ref[...], w2_ref[...], preferred_element_type=jnp.float32)
        + cnt_ref[...] * b2_ref[...]
    )
    ox_ref[...] = xn
    oa_ref[...] = (
        jnp.dot(xn, wa_ref[...], preferred_element_type=jnp.float32)
        + ba_ref[...]
    )
    ob_ref[...] = jnp.dot(
        xn, wb_ref[...], preferred_element_type=jnp.float32
    )


def _post_ab(s, w2_t, x, cnt_col, b2, wa_t, wb_t, ba, bm=400):
    M = x.shape[0]
    K = s.shape[1]
    N = w2_t.shape[1]
    return pl.pallas_call(
        _post_ab_kernel,
        grid=(M // bm,),
        in_specs=[
            pl.BlockSpec((bm, K), lambda i: (i, 0)),
            pl.BlockSpec((K, N), lambda i: (0, 0)),
            pl.BlockSpec((bm, N), lambda i: (i, 0)),
            pl.BlockSpec((bm, 1), lambda i: (i, 0)),
            pl.BlockSpec((1, N), lambda i: (0, 0)),
            pl.BlockSpec((N, N), lambda i: (0, 0)),
            pl.BlockSpec((N, N), lambda i: (0, 0)),
            pl.BlockSpec((1, N), lambda i: (0, 0)),
        ],
        out_specs=[
            pl.BlockSpec((bm, N), lambda i: (i, 0)),
            pl.BlockSpec((bm, N), lambda i: (i, 0)),
            pl.BlockSpec((bm, N), lambda i: (i, 0)),
        ],
        out_shape=[
            jax.ShapeDtypeStruct((M, N), jnp.float32),
            jax.ShapeDtypeStruct((M, N), jnp.float32),
            jax.ShapeDtypeStruct((M, N), jnp.float32),
        ],
    )(s, w2_t, x, cnt_col, b2[None, :], wa_t, wb_t, ba[None, :])


def _post_kernel(s_ref, w_ref, x_ref, cnt_ref, b_ref, o_ref):
    o_ref[...] = (
        x_ref[...]
        + jnp.dot(s_ref[...], w_ref[...], preferred_element_type=jnp.float32)
        + cnt_ref[...] * b_ref[...]
    )


def _post_matmul(s, w2_t, x, cnt_col, b2, bm=400):
    M = x.shape[0]
    K = s.shape[1]
    N = w2_t.shape[1]
    return pl.pallas_call(
        _post_kernel,
        grid=(M // bm,),
        in_specs=[
            pl.BlockSpec((bm, K), lambda i: (i, 0)),
            pl.BlockSpec((K, N), lambda i: (0, 0)),
            pl.BlockSpec((bm, N), lambda i: (i, 0)),
            pl.BlockSpec((bm, 1), lambda i: (i, 0)),
            pl.BlockSpec((1, N), lambda i: (0, 0)),
        ],
        out_specs=pl.BlockSpec((bm, N), lambda i: (i, 0)),
        out_shape=jax.ShapeDtypeStruct((M, N), jnp.float32),
    )(s, w2_t, x, cnt_col, b2[None, :])


def _post2_proj_kernel(sbu_ref, w2bu_ref, xbu_ref, cbu_ref, b2bu_ref,
                       std_ref, w2td_ref, xtd_ref, ctd_ref, b2td_ref,
                       pa_ref, pb_ref, pbias_ref, o_ref):
    xbu = (
        xbu_ref[...]
        + jnp.dot(sbu_ref[...], w2bu_ref[...],
                  preferred_element_type=jnp.float32)
        + cbu_ref[...] * b2bu_ref[...]
    )
    xtd = (
        xtd_ref[...]
        + jnp.dot(std_ref[...], w2td_ref[...],
                  preferred_element_type=jnp.float32)
        + ctd_ref[...] * b2td_ref[...]
    )
    o_ref[...] = (
        jnp.dot(xbu, pa_ref[...], preferred_element_type=jnp.float32)
        + jnp.dot(xtd, pb_ref[...], preferred_element_type=jnp.float32)
        + pbias_ref[...]
    )


def _post2_proj(s_bu, w2bu_t, x_bu, cbu, b2bu, s_td, w2td_t, x_td, ctd, b2td,
                pa_t, pb_t, pbias, bm=400):
    M, N = x_bu.shape
    row = lambda i: (i, 0)
    full = lambda i: (0, 0)
    return pl.pallas_call(
        _post2_proj_kernel,
        grid=(M // bm,),
        in_specs=[
            pl.BlockSpec((bm, N), row),
            pl.BlockSpec((N, N), full),
            pl.BlockSpec((bm, N), row),
            pl.BlockSpec((bm, 1), row),
            pl.BlockSpec((1, N), full),
            pl.BlockSpec((bm, N), row),
            pl.BlockSpec((N, N), full),
            pl.BlockSpec((bm, N), row),
            pl.BlockSpec((bm, 1), row),
            pl.BlockSpec((1, N), full),
            pl.BlockSpec((N, N), full),
            pl.BlockSpec((N, N), full),
            pl.BlockSpec((1, N), full),
        ],
        out_specs=pl.BlockSpec((bm, N), row),
        out_shape=jax.ShapeDtypeStruct((M, N), jnp.float32),
    )(s_bu, w2bu_t, x_bu, cbu, b2bu[None, :], s_td, w2td_t, x_td, ctd,
      b2td[None, :], pa_t, pb_t, pbias[None, :])


def _proj_kernel(a_ref, b_ref, wa_ref, wb_ref, bias_ref, o_ref):
    o_ref[...] = (
        jnp.dot(a_ref[...], wa_ref[...], preferred_element_type=jnp.float32)
        + jnp.dot(b_ref[...], wb_ref[...], preferred_element_type=jnp.float32)
        + bias_ref[...]
    )


def _final_proj(x_bu, x_td, wa_t, wb_t, bias, bm=400):
    M, K = x_bu.shape
    N = wa_t.shape[1]
    return pl.pallas_call(
        _proj_kernel,
        grid=(M // bm,),
        in_specs=[
            pl.BlockSpec((bm, K), lambda i: (i, 0)),
            pl.BlockSpec((bm, K), lambda i: (i, 0)),
            pl.BlockSpec((K, N), lambda i: (0, 0)),
            pl.BlockSpec((K, N), lambda i: (0, 0)),
            pl.BlockSpec((1, N), lambda i: (0, 0)),
        ],
        out_specs=pl.BlockSpec((bm, N), lambda i: (i, 0)),
        out_shape=jax.ShapeDtypeStruct((M, N), jnp.float32),
    )(x_bu, x_td, wa_t, wb_t, bias[None, :])


def kernel(edge_index, species_ids, branch_lengths, params):
    d = params["internal_embedding"].shape[0]

    x0 = jnp.take(params["species_embedding"], species_ids, axis=0)

    p2c = edge_index[:, 0::2]
    c2p = edge_index[:, 1::2]
    bl_p2c = branch_lengths[0::2]
    bl_c2p = branch_lengths[1::2]

    pad_i = jnp.zeros((_EP - _E,), jnp.int32)
    pad_t = jnp.full((_EP - _E,), _TRASH, jnp.int32)
    pad_f = jnp.zeros((_EP - _E,), jnp.float32)
    zrow = jnp.zeros((_NPAD,), jnp.float32)

    def prep_dir(ei, bl):
        src = jnp.concatenate([ei[0], pad_i])
        dst = jnp.concatenate([ei[1], pad_t])
        blp = jnp.concatenate([bl, pad_f])
        dst3d = dst.reshape(32, 1, _TILE_E)
        bl2 = blp.reshape(_EP, 1)
        cnt32 = _count_call(dst, zrow).reshape(32, _NPAD)
        cnt_col = _reduce_counts(cnt32).reshape(_NPAD, 1)
        return src, dst, bl2, dst3d, cnt_col

    ebu = prep_dir(c2p, bl_c2p)
    etd = prep_dir(p2c, bl_p2c)
    lbu0, lbu1 = params["bu"]
    ltd0, ltd1 = params["td"]
    a_bu, b_bu = _matmul_ab(
        x0, lbu0["W1"][:, :d].T, lbu0["W1"][:, d : 2 * d].T, lbu0["b1"]
    )
    a_td, b_td = _matmul_ab(
        x0, ltd0["W1"][:, :d].T, ltd0["W1"][:, d : 2 * d].T, ltd0["b1"]
    )
    ha_bu, hb_bu = _gatherh_call(a_bu, b_bu, ebu[0], ebu[1])
    ha_td, hb_td = _gatherh_call(a_td, b_td, etd[0], etd[1])
    s_bu = _scatter_call(ha_bu, hb_bu, ebu[2], lbu0["W1"][:, 2 * d], ebu[3])
    s_td = _scatter_call(ha_td, hb_td, etd[2], ltd0["W1"][:, 2 * d], etd[3])
    x_bu, a_bu, b_bu = _post_ab(
        s_bu, lbu0["W2"].T, x0, ebu[4], lbu0["b2"],
        lbu1["W1"][:, :d].T, lbu1["W1"][:, d : 2 * d].T, lbu1["b1"]
    )
    x_td, a_td, b_td = _post_ab(
        s_td, ltd0["W2"].T, x0, etd[4], ltd0["b2"],
        ltd1["W1"][:, :d].T, ltd1["W1"][:, d : 2 * d].T, ltd1["b1"]
    )
    ha_bu, hb_bu = _gatherh_call(a_bu, b_bu, ebu[0], ebu[1])
    ha_td, hb_td = _gatherh_call(a_td, b_td, etd[0], etd[1])
    s_bu = _scatter_call(ha_bu, hb_bu, ebu[2], lbu1["W1"][:, 2 * d], ebu[3])
    s_td = _scatter_call(ha_td, hb_td, etd[2], ltd1["W1"][:, 2 * d], etd[3])

    pw = params["proj_W"]
    return _post2_proj(
        s_bu, lbu1["W2"].T, x_bu, ebu[4], lbu1["b2"],
        s_td, ltd1["W2"].T, x_td, etd[4], ltd1["b2"],
        pw[:, :d].T, pw[:, d:].T, params["proj_b"],
    )


# speedup vs baseline: 1.0822x; 1.0822x over previous
import functools

import jax
import jax.numpy as jnp
from jax import lax
from jax.experimental import pallas as pl
from jax.experimental.pallas import tpu as pltpu
from jax.experimental.pallas import tpu_sc as plsc

_N = 50000
_D = 256
_E = 49999
_EP = 50176
_TILE_E = _EP // 32
_NGRP = _TILE_E // 16
_NPAD = 50176
_TRASH = 50100


def _gatherh_body(a_hbm, b_hbm, src_hbm, dst_hbm, bl_hbm, w1c_hbm, h_hbm,
                  src_s, dst_s, bl_s, a16a, b16a, a16b, b16b, h16, w1c_v,
                  sema, semb, sema2, semb2):
    c = lax.axis_index("c")
    s = lax.axis_index("s")
    w = s * 2 + c
    ebase = w * _TILE_E

    pltpu.sync_copy(src_hbm.at[pl.ds(ebase, _TILE_E)], src_s)
    pltpu.sync_copy(dst_hbm.at[pl.ds(ebase, _TILE_E)], dst_s)
    pltpu.sync_copy(bl_hbm.at[pl.ds(ebase, _TILE_E)], bl_s)
    pltpu.sync_copy(w1c_hbm, w1c_v)

    def _issue(g, abuf, bbuf, s1, s2):
        idxs = src_s[pl.ds(g * 16, 16)]
        idxg = dst_s[pl.ds(g * 16, 16)]
        cpa = pltpu.async_copy(a_hbm.at[idxs], abuf, s1)
        cpb = pltpu.async_copy(b_hbm.at[idxg], bbuf, s2)
        return cpa, cpb

    wvs = [w1c_v[pl.ds(cc * 16, 16)] for cc in range(16)]

    def _compute(g, abuf, bbuf):
        for j in range(16):
            blj = plsc.load_gather(
                bl_s, [jnp.full((16,), 0, jnp.int32) + (g * 16 + j)]
            )
            for cc in range(16):
                av = abuf[j, pl.ds(cc * 16, 16)]
                bv = bbuf[j, pl.ds(cc * 16, 16)]
                h16[j, pl.ds(cc * 16, 16)] = jnp.maximum(
                    av + bv + blj * wvs[cc], 0.0
                )
        pltpu.sync_copy(h16, h_hbm.at[pl.ds(ebase + g * 16, 16)])

    cpa0, cpb0 = _issue(0, a16a, b16a, sema, semb)

    def pair_body(it, _):
        g0 = 2 * it
        pltpu.make_async_copy(a_hbm.at[pl.ds(0, 16)], a16a, sema).wait()
        pltpu.make_async_copy(b_hbm.at[pl.ds(0, 16)], b16a, semb).wait()
        _issue(g0 + 1, a16b, b16b, sema2, semb2)
        _compute(g0, a16a, b16a)
        pltpu.make_async_copy(a_hbm.at[pl.ds(0, 16)], a16b, sema2).wait()
        pltpu.make_async_copy(b_hbm.at[pl.ds(0, 16)], b16b, semb2).wait()

        @pl.when(it + 1 < _NGRP // 2)
        def _():
            _issue(g0 + 2, a16a, b16a, sema, semb)

        _compute(g0 + 1, a16b, b16b)
        return 0

    lax.fori_loop(0, _NGRP // 2, pair_body, 0)


_gatherh_call = pl.kernel(
    _gatherh_body,
    out_type=jax.ShapeDtypeStruct((_EP, _D), jnp.float32),
    mesh=plsc.VectorSubcoreMesh(
        core_axis_name="c", subcore_axis_name="s", num_cores=2, num_subcores=16
    ),
    compiler_params=pltpu.CompilerParams(needs_layout_passes=False),
    scratch_types=[
        pltpu.VMEM((_TILE_E,), jnp.int32),
        pltpu.VMEM((_TILE_E,), jnp.int32),
        pltpu.VMEM((_TILE_E,), jnp.float32),
        pltpu.VMEM((16, _D), jnp.float32),
        pltpu.VMEM((16, _D), jnp.float32),
        pltpu.VMEM((16, _D), jnp.float32),
        pltpu.VMEM((16, _D), jnp.float32),
        pltpu.VMEM((16, _D), jnp.float32),
        pltpu.VMEM((_D,), jnp.float32),
        pltpu.SemaphoreType.DMA,
        pltpu.SemaphoreType.DMA,
        pltpu.SemaphoreType.DMA,
        pltpu.SemaphoreType.DMA,
    ],
)


def _count_body(dst_hbm, z_hbm, cnt_hbm, dst_s, cl):
    c = lax.axis_index("c")
    s = lax.axis_index("s")
    w = s * 2 + c
    ebase = w * _TILE_E

    pltpu.sync_copy(dst_hbm.at[pl.ds(ebase, _TILE_E)], dst_s)
    pltpu.sync_copy(z_hbm, cl)
    lanes = lax.iota(jnp.int32, 16)

    def _lane_take(v, idx):
        return lax.gather(
            v,
            idx[:, None],
            lax.GatherDimensionNumbers(
                offset_dims=(),
                collapsed_slice_dims=(0,),
                start_index_map=(0,),
            ),
            (1,),
            mode=lax.GatherScatterMode.PROMISE_IN_BOUNDS,
        )

    def chunk_body(i, _):
        d16 = dst_s[pl.ds(i * 16, 16)]
        later = jnp.zeros((16,), jnp.int32)
        earlier = jnp.zeros((16,), jnp.int32)
        for sh in range(1, 16):
            up = _lane_take(d16, jnp.minimum(lanes + sh, 15))
            later = later + jnp.where(
                (lanes + sh <= 15) & (d16 == up), 1, 0
            )
            dn = _lane_take(d16, jnp.maximum(lanes - sh, 0))
            earlier = earlier + jnp.where(
                (lanes - sh >= 0) & (d16 == dn), 1, 0
            )
        rep = earlier == 0
        total = (later + 1).astype(jnp.float32)
        plsc.addupdate_scatter(cl, [d16], total, mask=rep)
        return 0

    lax.fori_loop(0, _NGRP, chunk_body, 0)
    pltpu.sync_copy(cl, cnt_hbm.at[pl.ds(w * _NPAD, _NPAD)])


_count_call = pl.kernel(
    _count_body,
    out_type=jax.ShapeDtypeStruct((32 * _NPAD,), jnp.float32),
    mesh=plsc.VectorSubcoreMesh(
        core_axis_name="c", subcore_axis_name="s", num_cores=2, num_subcores=16
    ),
    compiler_params=pltpu.CompilerParams(needs_layout_passes=False),
    scratch_types=[
        pltpu.VMEM((_TILE_E,), jnp.int32),
        pltpu.VMEM((_NPAD,), jnp.float32),
    ],
)


def _scatter_kernel(h_ref, dst_ref, o_ref):
    @pl.when(pl.program_id(0) == 0)
    def _():
        o_ref[...] = jnp.zeros((_NPAD, _D), jnp.float32)

    def eb(j8, _):
        for u in range(16):
            j = j8 * 16 + u
            d = dst_ref[0, 0, j]
            o_ref[pl.ds(d, 1), :] += h_ref[pl.ds(j, 1), :]
        return 0

    lax.fori_loop(0, _TILE_E // 16, eb, 0)


def _scatter_call(h2, dst3d):
    return pl.pallas_call(
        _scatter_kernel,
        grid=(_EP // _TILE_E,),
        in_specs=[
            pl.BlockSpec((_TILE_E, _D), lambda i: (i, 0)),
            pl.BlockSpec((1, 1, _TILE_E), lambda i: (i, 0, 0),
                         memory_space=pltpu.SMEM),
        ],
        out_specs=pl.BlockSpec((_NPAD, _D), lambda i: (0, 0)),
        out_shape=jax.ShapeDtypeStruct((_NPAD, _D), jnp.float32),
        compiler_params=pltpu.CompilerParams(
            vmem_limit_bytes=60 * 1024 * 1024,
        ),
    )(h2, dst3d)


def _reduce_kernel(c_ref, o_ref):
    o_ref[...] = jnp.sum(c_ref[...], axis=0, keepdims=True)


def _reduce_counts(cnt32, bn=6272):
    return pl.pallas_call(
        _reduce_kernel,
        grid=(_NPAD // bn,),
        in_specs=[pl.BlockSpec((32, bn), lambda i: (0, i))],
        out_specs=pl.BlockSpec((1, bn), lambda i: (0, i)),
        out_shape=jax.ShapeDtypeStruct((1, _NPAD), jnp.float32),
    )(cnt32)


def _mm2_kernel(x_ref, wa_ref, wb_ref, ba_ref, oa_ref, ob_ref):
    oa_ref[...] = (
        jnp.dot(x_ref[...], wa_ref[...], preferred_element_type=jnp.float32)
        + ba_ref[...]
    )
    ob_ref[...] = jnp.dot(
        x_ref[...], wb_ref[...], preferred_element_type=jnp.float32
    )


def _matmul_ab(x, wa_t, wb_t, ba, bm=400):
    M, K = x.shape
    N = wa_t.shape[1]
    return pl.pallas_call(
        _mm2_kernel,
        grid=(M // bm,),
        in_specs=[
            pl.BlockSpec((bm, K), lambda i: (i, 0)),
            pl.BlockSpec((K, N), lambda i: (0, 0)),
            pl.BlockSpec((K, N), lambda i: (0, 0)),
            pl.BlockSpec((1, N), lambda i: (0, 0)),
        ],
        out_specs=[
            pl.BlockSpec((bm, N), lambda i: (i, 0)),
            pl.BlockSpec((bm, N), lambda i: (i, 0)),
        ],
        out_shape=[
            jax.ShapeDtypeStruct((M, N), jnp.float32),
            jax.ShapeDtypeStruct((M, N), jnp.float32),
        ],
    )(x, wa_t, wb_t, ba[None, :])


def _post_ab_kernel(s_ref, w2_ref, x_ref, cnt_ref, b2_ref, wa_ref, wb_ref,
                    ba_ref, ox_ref, oa_ref, ob_ref):
    xn = (
        x_ref[...]
        + jnp.dot(s_ref[...], w2_ref[...], preferred_element_type=jnp.float32)
        + cnt_ref[...] * b2_ref[...]
    )
    ox_ref[...] = xn
    oa_ref[...] = (
        jnp.dot(xn, wa_ref[...], preferred_element_type=jnp.float32)
        + ba_ref[...]
    )
    ob_ref[...] = jnp.dot(
        xn, wb_ref[...], preferred_element_type=jnp.float32
    )


def _post_ab(s, w2_t, x, cnt_col, b2, wa_t, wb_t, ba, bm=400):
    M = x.shape[0]
    K = s.shape[1]
    N = w2_t.shape[1]
    return pl.pallas_call(
        _post_ab_kernel,
        grid=(M // bm,),
        in_specs=[
            pl.BlockSpec((bm, K), lambda i: (i, 0)),
            pl.BlockSpec((K, N), lambda i: (0, 0)),
            pl.BlockSpec((bm, N), lambda i: (i, 0)),
            pl.BlockSpec((bm, 1), lambda i: (i, 0)),
            pl.BlockSpec((1, N), lambda i: (0, 0)),
            pl.BlockSpec((N, N), lambda i: (0, 0)),
            pl.BlockSpec((N, N), lambda i: (0, 0)),
            pl.BlockSpec((1, N), lambda i: (0, 0)),
        ],
        out_specs=[
            pl.BlockSpec((bm, N), lambda i: (i, 0)),
            pl.BlockSpec((bm, N), lambda i: (i, 0)),
            pl.BlockSpec((bm, N), lambda i: (i, 0)),
        ],
        out_shape=[
            jax.ShapeDtypeStruct((M, N), jnp.float32),
            jax.ShapeDtypeStruct((M, N), jnp.float32),
            jax.ShapeDtypeStruct((M, N), jnp.float32),
        ],
    )(s, w2_t, x, cnt_col, b2[None, :], wa_t, wb_t, ba[None, :])


def _post_kernel(s_ref, w_ref, x_ref, cnt_ref, b_ref, o_ref):
    o_ref[...] = (
        x_ref[...]
        + jnp.dot(s_ref[...], w_ref[...], preferred_element_type=jnp.float32)
        + cnt_ref[...] * b_ref[...]
    )


def _post_matmul(s, w2_t, x, cnt_col, b2, bm=400):
    M = x.shape[0]
    K = s.shape[1]
    N = w2_t.shape[1]
    return pl.pallas_call(
        _post_kernel,
        grid=(M // bm,),
        in_specs=[
            pl.BlockSpec((bm, K), lambda i: (i, 0)),
            pl.BlockSpec((K, N), lambda i: (0, 0)),
            pl.BlockSpec((bm, N), lambda i: (i, 0)),
            pl.BlockSpec((bm, 1), lambda i: (i, 0)),
            pl.BlockSpec((1, N), lambda i: (0, 0)),
        ],
        out_specs=pl.BlockSpec((bm, N), lambda i: (i, 0)),
        out_shape=jax.ShapeDtypeStruct((M, N), jnp.float32),
    )(s, w2_t, x, cnt_col, b2[None, :])


def _post2_proj_kernel(sbu_ref, w2bu_ref, xbu_ref, cbu_ref, b2bu_ref,
                       std_ref, w2td_ref, xtd_ref, ctd_ref, b2td_ref,
                       pa_ref, pb_ref, pbias_ref, o_ref):
    xbu = (
        xbu_ref[...]
        + jnp.dot(sbu_ref[...], w2bu_ref[...],
                  preferred_element_type=jnp.float32)
        + cbu_ref[...] * b2bu_ref[...]
    )
    xtd = (
        xtd_ref[...]
        + jnp.dot(std_ref[...], w2td_ref[...],
                  preferred_element_type=jnp.float32)
        + ctd_ref[...] * b2td_ref[...]
    )
    o_ref[...] = (
        jnp.dot(xbu, pa_ref[...], preferred_element_type=jnp.float32)
        + jnp.dot(xtd, pb_ref[...], preferred_element_type=jnp.float32)
        + pbias_ref[...]
    )


def _post2_proj(s_bu, w2bu_t, x_bu, cbu, b2bu, s_td, w2td_t, x_td, ctd, b2td,
                pa_t, pb_t, pbias, bm=400):
    M, N = x_bu.shape
    row = lambda i: (i, 0)
    full = lambda i: (0, 0)
    return pl.pallas_call(
        _post2_proj_kernel,
        grid=(M // bm,),
        in_specs=[
            pl.BlockSpec((bm, N), row),
            pl.BlockSpec((N, N), full),
            pl.BlockSpec((bm, N), row),
            pl.BlockSpec((bm, 1), row),
            pl.BlockSpec((1, N), full),
            pl.BlockSpec((bm, N), row),
            pl.BlockSpec((N, N), full),
            pl.BlockSpec((bm, N), row),
            pl.BlockSpec((bm, 1), row),
            pl.BlockSpec((1, N), full),
            pl.BlockSpec((N, N), full),
            pl.BlockSpec((N, N), full),
            pl.BlockSpec((1, N), full),
        ],
        out_specs=pl.BlockSpec((bm, N), row),
        out_shape=jax.ShapeDtypeStruct((M, N), jnp.float32),
    )(s_bu, w2bu_t, x_bu, cbu, b2bu[None, :], s_td, w2td_t, x_td, ctd,
      b2td[None, :], pa_t, pb_t, pbias[None, :])


def _proj_kernel(a_ref, b_ref, wa_ref, wb_ref, bias_ref, o_ref):
    o_ref[...] = (
        jnp.dot(a_ref[...], wa_ref[...], preferred_element_type=jnp.float32)
        + jnp.dot(b_ref[...], wb_ref[...], preferred_element_type=jnp.float32)
        + bias_ref[...]
    )


def _final_proj(x_bu, x_td, wa_t, wb_t, bias, bm=400):
    M, K = x_bu.shape
    N = wa_t.shape[1]
    return pl.pallas_call(
        _proj_kernel,
        grid=(M // bm,),
        in_specs=[
            pl.BlockSpec((bm, K), lambda i: (i, 0)),
            pl.BlockSpec((bm, K), lambda i: (i, 0)),
            pl.BlockSpec((K, N), lambda i: (0, 0)),
            pl.BlockSpec((K, N), lambda i: (0, 0)),
            pl.BlockSpec((1, N), lambda i: (0, 0)),
        ],
        out_specs=pl.BlockSpec((bm, N), lambda i: (i, 0)),
        out_shape=jax.ShapeDtypeStruct((M, N), jnp.float32),
    )(x_bu, x_td, wa_t, wb_t, bias[None, :])


def kernel(edge_index, species_ids, branch_lengths, params):
    d = params["internal_embedding"].shape[0]

    x0 = jnp.take(params["species_embedding"], species_ids, axis=0)

    p2c = edge_index[:, 0::2]
    c2p = edge_index[:, 1::2]
    bl_p2c = branch_lengths[0::2]
    bl_c2p = branch_lengths[1::2]

    pad_i = jnp.zeros((_EP - _E,), jnp.int32)
    pad_t = jnp.full((_EP - _E,), _TRASH, jnp.int32)
    pad_f = jnp.zeros((_EP - _E,), jnp.float32)
    zrow = jnp.zeros((_NPAD,), jnp.float32)

    def prep_dir(ei, bl):
        src = jnp.concatenate([ei[0], pad_i])
        dst = jnp.concatenate([ei[1], pad_t])
        blp = jnp.concatenate([bl, pad_f])
        dst3d = dst.reshape(32, 1, _TILE_E)
        cnt32 = _count_call(dst, zrow).reshape(32, _NPAD)
        cnt_col = _reduce_counts(cnt32).reshape(_NPAD, 1)
        return src, dst, blp, dst3d, cnt_col

    ebu = prep_dir(c2p, bl_c2p)
    etd = prep_dir(p2c, bl_p2c)
    lbu0, lbu1 = params["bu"]
    ltd0, ltd1 = params["td"]
    a_bu, b_bu = _matmul_ab(
        x0, lbu0["W1"][:, :d].T, lbu0["W1"][:, d : 2 * d].T, lbu0["b1"]
    )
    a_td, b_td = _matmul_ab(
        x0, ltd0["W1"][:, :d].T, ltd0["W1"][:, d : 2 * d].T, ltd0["b1"]
    )
    h_bu = _gatherh_call(a_bu, b_bu, ebu[0], ebu[1], ebu[2],
                         lbu0["W1"][:, 2 * d])
    h_td = _gatherh_call(a_td, b_td, etd[0], etd[1], etd[2],
                         ltd0["W1"][:, 2 * d])
    s_bu = _scatter_call(h_bu, ebu[3])
    s_td = _scatter_call(h_td, etd[3])
    x_bu, a_bu, b_bu = _post_ab(
        s_bu, lbu0["W2"].T, x0, ebu[4], lbu0["b2"],
        lbu1["W1"][:, :d].T, lbu1["W1"][:, d : 2 * d].T, lbu1["b1"]
    )
    x_td, a_td, b_td = _post_ab(
        s_td, ltd0["W2"].T, x0, etd[4], ltd0["b2"],
        ltd1["W1"][:, :d].T, ltd1["W1"][:, d : 2 * d].T, ltd1["b1"]
    )
    h_bu = _gatherh_call(a_bu, b_bu, ebu[0], ebu[1], ebu[2],
                         lbu1["W1"][:, 2 * d])
    h_td = _gatherh_call(a_td, b_td, etd[0], etd[1], etd[2],
                         ltd1["W1"][:, 2 * d])
    s_bu = _scatter_call(h_bu, ebu[3])
    s_td = _scatter_call(h_td, etd[3])

    pw = params["proj_W"]
    return _post2_proj(
        s_bu, lbu1["W2"].T, x_bu, ebu[4], lbu1["b2"],
        s_td, ltd1["W2"].T, x_td, etd[4], ltd1["b2"],
        pw[:, :d].T, pw[:, d:].T, params["proj_b"],
    )


# speedup vs baseline: 1.0825x; 1.0003x over previous
import functools

import jax
import jax.numpy as jnp
from jax import lax
from jax.experimental import pallas as pl
from jax.experimental.pallas import tpu as pltpu
from jax.experimental.pallas import tpu_sc as plsc

_N = 50000
_D = 256
_E = 49999
_EP = 50176
_TILE_E = _EP // 32
_NGRP = _TILE_E // 16
_NPAD = 50176
_TRASH = 50100


def _gatherh_body(a_hbm, b_hbm, src_hbm, dst_hbm, bl_hbm, w1c_hbm, h_hbm,
                  src_s, dst_s, bl_s, a16a, b16a, a16b, b16b, h16a, h16b,
                  w1c_v, sema, semb, sema2, semb2, semha, semhb):
    c = lax.axis_index("c")
    s = lax.axis_index("s")
    w = s * 2 + c
    ebase = w * _TILE_E

    pltpu.sync_copy(src_hbm.at[pl.ds(ebase, _TILE_E)], src_s)
    pltpu.sync_copy(dst_hbm.at[pl.ds(ebase, _TILE_E)], dst_s)
    pltpu.sync_copy(bl_hbm.at[pl.ds(ebase, _TILE_E)], bl_s)
    pltpu.sync_copy(w1c_hbm, w1c_v)

    def _issue(g, abuf, bbuf, s1, s2):
        idxs = src_s[pl.ds(g * 16, 16)]
        idxg = dst_s[pl.ds(g * 16, 16)]
        cpa = pltpu.async_copy(a_hbm.at[idxs], abuf, s1)
        cpb = pltpu.async_copy(b_hbm.at[idxg], bbuf, s2)
        return cpa, cpb

    wvs = [w1c_v[pl.ds(cc * 16, 16)] for cc in range(16)]

    def _compute(g, abuf, bbuf, hbuf, hsem, it):
        @pl.when(it > 0)
        def _():
            pltpu.make_async_copy(
                hbuf, h_hbm.at[pl.ds(ebase, 16)], hsem
            ).wait()
        for j in range(16):
            blj = plsc.load_gather(
                bl_s, [jnp.full((16,), 0, jnp.int32) + (g * 16 + j)]
            )
            for cc in range(16):
                av = abuf[j, pl.ds(cc * 16, 16)]
                bv = bbuf[j, pl.ds(cc * 16, 16)]
                hbuf[j, pl.ds(cc * 16, 16)] = jnp.maximum(
                    av + bv + blj * wvs[cc], 0.0
                )
        pltpu.async_copy(hbuf, h_hbm.at[pl.ds(ebase + g * 16, 16)], hsem)

    cpa0, cpb0 = _issue(0, a16a, b16a, sema, semb)

    def pair_body(it, _):
        g0 = 2 * it
        pltpu.make_async_copy(a_hbm.at[pl.ds(0, 16)], a16a, sema).wait()
        pltpu.make_async_copy(b_hbm.at[pl.ds(0, 16)], b16a, semb).wait()
        _issue(g0 + 1, a16b, b16b, sema2, semb2)
        _compute(g0, a16a, b16a, h16a, semha, it)
        pltpu.make_async_copy(a_hbm.at[pl.ds(0, 16)], a16b, sema2).wait()
        pltpu.make_async_copy(b_hbm.at[pl.ds(0, 16)], b16b, semb2).wait()

        @pl.when(it + 1 < _NGRP // 2)
        def _():
            _issue(g0 + 2, a16a, b16a, sema, semb)

        _compute(g0 + 1, a16b, b16b, h16b, semhb, it)
        return 0

    lax.fori_loop(0, _NGRP // 2, pair_body, 0)
    pltpu.make_async_copy(h16a, h_hbm.at[pl.ds(ebase, 16)], semha).wait()
    pltpu.make_async_copy(h16b, h_hbm.at[pl.ds(ebase, 16)], semhb).wait()


_gatherh_call = pl.kernel(
    _gatherh_body,
    out_type=jax.ShapeDtypeStruct((_EP, _D), jnp.float32),
    mesh=plsc.VectorSubcoreMesh(
        core_axis_name="c", subcore_axis_name="s", num_cores=2, num_subcores=16
    ),
    compiler_params=pltpu.CompilerParams(needs_layout_passes=False),
    scratch_types=[
        pltpu.VMEM((_TILE_E,), jnp.int32),
        pltpu.VMEM((_TILE_E,), jnp.int32),
        pltpu.VMEM((_TILE_E,), jnp.float32),
        pltpu.VMEM((16, _D), jnp.float32),
        pltpu.VMEM((16, _D), jnp.float32),
        pltpu.VMEM((16, _D), jnp.float32),
        pltpu.VMEM((16, _D), jnp.float32),
        pltpu.VMEM((16, _D), jnp.float32),
        pltpu.VMEM((16, _D), jnp.float32),
        pltpu.VMEM((_D,), jnp.float32),
        pltpu.SemaphoreType.DMA,
        pltpu.SemaphoreType.DMA,
        pltpu.SemaphoreType.DMA,
        pltpu.SemaphoreType.DMA,
        pltpu.SemaphoreType.DMA,
        pltpu.SemaphoreType.DMA,
    ],
)


def _count_body(dst_hbm, z_hbm, cnt_hbm, dst_s, cl):
    c = lax.axis_index("c")
    s = lax.axis_index("s")
    w = s * 2 + c
    ebase = w * _TILE_E

    pltpu.sync_copy(dst_hbm.at[pl.ds(ebase, _TILE_E)], dst_s)
    pltpu.sync_copy(z_hbm, cl)
    lanes = lax.iota(jnp.int32, 16)

    def _lane_take(v, idx):
        return lax.gather(
            v,
            idx[:, None],
            lax.GatherDimensionNumbers(
                offset_dims=(),
                collapsed_slice_dims=(0,),
                start_index_map=(0,),
            ),
            (1,),
            mode=lax.GatherScatterMode.PROMISE_IN_BOUNDS,
        )

    def chunk_body(i, _):
        d16 = dst_s[pl.ds(i * 16, 16)]
        later = jnp.zeros((16,), jnp.int32)
        earlier = jnp.zeros((16,), jnp.int32)
        for sh in range(1, 16):
            up = _lane_take(d16, jnp.minimum(lanes + sh, 15))
            later = later + jnp.where(
                (lanes + sh <= 15) & (d16 == up), 1, 0
            )
            dn = _lane_take(d16, jnp.maximum(lanes - sh, 0))
            earlier = earlier + jnp.where(
                (lanes - sh >= 0) & (d16 == dn), 1, 0
            )
        rep = earlier == 0
        total = (later + 1).astype(jnp.float32)
        plsc.addupdate_scatter(cl, [d16], total, mask=rep)
        return 0

    lax.fori_loop(0, _NGRP, chunk_body, 0)
    pltpu.sync_copy(cl, cnt_hbm.at[pl.ds(w * _NPAD, _NPAD)])


_count_call = pl.kernel(
    _count_body,
    out_type=jax.ShapeDtypeStruct((32 * _NPAD,), jnp.float32),
    mesh=plsc.VectorSubcoreMesh(
        core_axis_name="c", subcore_axis_name="s", num_cores=2, num_subcores=16
    ),
    compiler_params=pltpu.CompilerParams(needs_layout_passes=False),
    scratch_types=[
        pltpu.VMEM((_TILE_E,), jnp.int32),
        pltpu.VMEM((_NPAD,), jnp.float32),
    ],
)


def _scatter_kernel(h_ref, dst_ref, o_ref):
    @pl.when(pl.program_id(0) == 0)
    def _():
        o_ref[...] = jnp.zeros((_NPAD, _D), jnp.float32)

    def eb(j8, _):
        for u in range(16):
            j = j8 * 16 + u
            d = dst_ref[0, 0, j]
            o_ref[pl.ds(d, 1), :] += h_ref[pl.ds(j, 1), :]
        return 0

    lax.fori_loop(0, _TILE_E // 16, eb, 0)


def _scatter_call(h2, dst3d):
    return pl.pallas_call(
        _scatter_kernel,
        grid=(_EP // _TILE_E,),
        in_specs=[
            pl.BlockSpec((_TILE_E, _D), lambda i: (i, 0)),
            pl.BlockSpec((1, 1, _TILE_E), lambda i: (i, 0, 0),
                         memory_space=pltpu.SMEM),
        ],
        out_specs=pl.BlockSpec((_NPAD, _D), lambda i: (0, 0)),
        out_shape=jax.ShapeDtypeStruct((_NPAD, _D), jnp.float32),
        compiler_params=pltpu.CompilerParams(
            vmem_limit_bytes=60 * 1024 * 1024,
        ),
    )(h2, dst3d)


def _reduce_kernel(c_ref, o_ref):
    o_ref[...] = jnp.sum(c_ref[...], axis=0, keepdims=True)


def _reduce_counts(cnt32, bn=6272):
    return pl.pallas_call(
        _reduce_kernel,
        grid=(_NPAD // bn,),
        in_specs=[pl.BlockSpec((32, bn), lambda i: (0, i))],
        out_specs=pl.BlockSpec((1, bn), lambda i: (0, i)),
        out_shape=jax.ShapeDtypeStruct((1, _NPAD), jnp.float32),
    )(cnt32)


def _mm2_kernel(x_ref, wa_ref, wb_ref, ba_ref, oa_ref, ob_ref):
    oa_ref[...] = (
        jnp.dot(x_ref[...], wa_ref[...], preferred_element_type=jnp.float32)
        + ba_ref[...]
    )
    ob_ref[...] = jnp.dot(
        x_ref[...], wb_ref[...], preferred_element_type=jnp.float32
    )


def _matmul_ab(x, wa_t, wb_t, ba, bm=400):
    M, K = x.shape
    N = wa_t.shape[1]
    return pl.pallas_call(
        _mm2_kernel,
        grid=(M // bm,),
        in_specs=[
            pl.BlockSpec((bm, K), lambda i: (i, 0)),
            pl.BlockSpec((K, N), lambda i: (0, 0)),
            pl.BlockSpec((K, N), lambda i: (0, 0)),
            pl.BlockSpec((1, N), lambda i: (0, 0)),
        ],
        out_specs=[
            pl.BlockSpec((bm, N), lambda i: (i, 0)),
            pl.BlockSpec((bm, N), lambda i: (i, 0)),
        ],
        out_shape=[
            jax.ShapeDtypeStruct((M, N), jnp.float32),
            jax.ShapeDtypeStruct((M, N), jnp.float32),
        ],
    )(x, wa_t, wb_t, ba[None, :])


def _post_ab_kernel(s_ref, w2_ref, x_ref, cnt_ref, b2_ref, wa_ref, wb_ref,
                    ba_ref, ox_ref, oa_ref, ob_ref):
    xn = (
        x_ref[...]
        + jnp.dot(s_ref[...], w2_ref[...], preferred_element_type=jnp.float32)
        + cnt_ref[...] * b2_ref[...]
    )
    ox_ref[...] = xn
    oa_ref[...] = (
        jnp.dot(xn, wa_ref[...], preferred_element_type=jnp.float32)
        + ba_ref[...]
    )
    ob_ref[...] = jnp.dot(
        xn, wb_ref[...], preferred_element_type=jnp.float32
    )


def _post_ab(s, w2_t, x, cnt_col, b2, wa_t, wb_t, ba, bm=400):
    M = x.shape[0]
    K = s.shape[1]
    N = w2_t.shape[1]
    return pl.pallas_call(
        _post_ab_kernel,
        grid=(M // bm,),
        in_specs=[
            pl.BlockSpec((bm, K), lambda i: (i, 0)),
            pl.BlockSpec((K, N), lambda i: (0, 0)),
            pl.BlockSpec((bm, N), lambda i: (i, 0)),
            pl.BlockSpec((bm, 1), lambda i: (i, 0)),
            pl.BlockSpec((1, N), lambda i: (0, 0)),
            pl.BlockSpec((N, N), lambda i: (0, 0)),
            pl.BlockSpec((N, N), lambda i: (0, 0)),
            pl.BlockSpec((1, N), lambda i: (0, 0)),
        ],
        out_specs=[
            pl.BlockSpec((bm, N), lambda i: (i, 0)),
            pl.BlockSpec((bm, N), lambda i: (i, 0)),
            pl.BlockSpec((bm, N), lambda i: (i, 0)),
        ],
        out_shape=[
            jax.ShapeDtypeStruct((M, N), jnp.float32),
            jax.ShapeDtypeStruct((M, N), jnp.float32),
            jax.ShapeDtypeStruct((M, N), jnp.float32),
        ],
    )(s, w2_t, x, cnt_col, b2[None, :], wa_t, wb_t, ba[None, :])


def _post_kernel(s_ref, w_ref, x_ref, cnt_ref, b_ref, o_ref):
    o_ref[...] = (
        x_ref[...]
        + jnp.dot(s_ref[...], w_ref[...], preferred_element_type=jnp.float32)
        + cnt_ref[...] * b_ref[...]
    )


def _post_matmul(s, w2_t, x, cnt_col, b2, bm=400):
    M = x.shape[0]
    K = s.shape[1]
    N = w2_t.shape[1]
    return pl.pallas_call(
        _post_kernel,
        grid=(M // bm,),
        in_specs=[
            pl.BlockSpec((bm, K), lambda i: (i, 0)),
            pl.BlockSpec((K, N), lambda i: (0, 0)),
            pl.BlockSpec((bm, N), lambda i: (i, 0)),
            pl.BlockSpec((bm, 1), lambda i: (i, 0)),
            pl.BlockSpec((1, N), lambda i: (0, 0)),
        ],
        out_specs=pl.BlockSpec((bm, N), lambda i: (i, 0)),
        out_shape=jax.ShapeDtypeStruct((M, N), jnp.float32),
    )(s, w2_t, x, cnt_col, b2[None, :])


def _post2_proj_kernel(sbu_ref, w2bu_ref, xbu_ref, cbu_ref, b2bu_ref,
                       std_ref, w2td_ref, xtd_ref, ctd_ref, b2td_ref,
                       pa_ref, pb_ref, pbias_ref, o_ref):
    xbu = (
        xbu_ref[...]
        + jnp.dot(sbu_ref[...], w2bu_ref[...],
                  preferred_element_type=jnp.float32)
        + cbu_ref[...] * b2bu_ref[...]
    )
    xtd = (
        xtd_ref[...]
        + jnp.dot(std_ref[...], w2td_ref[...],
                  preferred_element_type=jnp.float32)
        + ctd_ref[...] * b2td_ref[...]
    )
    o_ref[...] = (
        jnp.dot(xbu, pa_ref[...], preferred_element_type=jnp.float32)
        + jnp.dot(xtd, pb_ref[...], preferred_element_type=jnp.float32)
        + pbias_ref[...]
    )


def _post2_proj(s_bu, w2bu_t, x_bu, cbu, b2bu, s_td, w2td_t, x_td, ctd, b2td,
                pa_t, pb_t, pbias, bm=400):
    M, N = x_bu.shape
    row = lambda i: (i, 0)
    full = lambda i: (0, 0)
    return pl.pallas_call(
        _post2_proj_kernel,
        grid=(M // bm,),
        in_specs=[
            pl.BlockSpec((bm, N), row),
            pl.BlockSpec((N, N), full),
            pl.BlockSpec((bm, N), row),
            pl.BlockSpec((bm, 1), row),
            pl.BlockSpec((1, N), full),
            pl.BlockSpec((bm, N), row),
            pl.BlockSpec((N, N), full),
            pl.BlockSpec((bm, N), row),
            pl.BlockSpec((bm, 1), row),
            pl.BlockSpec((1, N), full),
            pl.BlockSpec((N, N), full),
            pl.BlockSpec((N, N), full),
            pl.BlockSpec((1, N), full),
        ],
        out_specs=pl.BlockSpec((bm, N), row),
        out_shape=jax.ShapeDtypeStruct((M, N), jnp.float32),
    )(s_bu, w2bu_t, x_bu, cbu, b2bu[None, :], s_td, w2td_t, x_td, ctd,
      b2td[None, :], pa_t, pb_t, pbias[None, :])


def _proj_kernel(a_ref, b_ref, wa_ref, wb_ref, bias_ref, o_ref):
    o_ref[...] = (
        jnp.dot(a_ref[...], wa_ref[...], preferred_element_type=jnp.float32)
        + jnp.dot(b_ref[...], wb_ref[...], preferred_element_type=jnp.float32)
        + bias_ref[...]
    )


def _final_proj(x_bu, x_td, wa_t, wb_t, bias, bm=400):
    M, K = x_bu.shape
    N = wa_t.shape[1]
    return pl.pallas_call(
        _proj_kernel,
        grid=(M // bm,),
        in_specs=[
            pl.BlockSpec((bm, K), lambda i: (i, 0)),
            pl.BlockSpec((bm, K), lambda i: (i, 0)),
            pl.BlockSpec((K, N), lambda i: (0, 0)),
            pl.BlockSpec((K, N), lambda i: (0, 0)),
            pl.BlockSpec((1, N), lambda i: (0, 0)),
        ],
        out_specs=pl.BlockSpec((bm, N), lambda i: (i, 0)),
        out_shape=jax.ShapeDtypeStruct((M, N), jnp.float32),
    )(x_bu, x_td, wa_t, wb_t, bias[None, :])


def kernel(edge_index, species_ids, branch_lengths, params):
    d = params["internal_embedding"].shape[0]

    x0 = jnp.take(params["species_embedding"], species_ids, axis=0)

    p2c = edge_index[:, 0::2]
    c2p = edge_index[:, 1::2]
    bl_p2c = branch_lengths[0::2]
    bl_c2p = branch_lengths[1::2]

    pad_i = jnp.zeros((_EP - _E,), jnp.int32)
    pad_t = jnp.full((_EP - _E,), _TRASH, jnp.int32)
    pad_f = jnp.zeros((_EP - _E,), jnp.float32)
    zrow = jnp.zeros((_NPAD,), jnp.float32)

    def prep_dir(ei, bl):
        src = jnp.concatenate([ei[0], pad_i])
        dst = jnp.concatenate([ei[1], pad_t])
        blp = jnp.concatenate([bl, pad_f])
        dst3d = dst.reshape(32, 1, _TILE_E)
        cnt32 = _count_call(dst, zrow).reshape(32, _NPAD)
        cnt_col = _reduce_counts(cnt32).reshape(_NPAD, 1)
        return src, dst, blp, dst3d, cnt_col

    ebu = prep_dir(c2p, bl_c2p)
    etd = prep_dir(p2c, bl_p2c)
    lbu0, lbu1 = params["bu"]
    ltd0, ltd1 = params["td"]
    a_bu, b_bu = _matmul_ab(
        x0, lbu0["W1"][:, :d].T, lbu0["W1"][:, d : 2 * d].T, lbu0["b1"]
    )
    a_td, b_td = _matmul_ab(
        x0, ltd0["W1"][:, :d].T, ltd0["W1"][:, d : 2 * d].T, ltd0["b1"]
    )
    h_bu = _gatherh_call(a_bu, b_bu, ebu[0], ebu[1], ebu[2],
                         lbu0["W1"][:, 2 * d])
    h_td = _gatherh_call(a_td, b_td, etd[0], etd[1], etd[2],
                         ltd0["W1"][:, 2 * d])
    s_bu = _scatter_call(h_bu, ebu[3])
    s_td = _scatter_call(h_td, etd[3])
    x_bu, a_bu, b_bu = _post_ab(
        s_bu, lbu0["W2"].T, x0, ebu[4], lbu0["b2"],
        lbu1["W1"][:, :d].T, lbu1["W1"][:, d : 2 * d].T, lbu1["b1"]
    )
    x_td, a_td, b_td = _post_ab(
        s_td, ltd0["W2"].T, x0, etd[4], ltd0["b2"],
        ltd1["W1"][:, :d].T, ltd1["W1"][:, d : 2 * d].T, ltd1["b1"]
    )
    h_bu = _gatherh_call(a_bu, b_bu, ebu[0], ebu[1], ebu[2],
                         lbu1["W1"][:, 2 * d])
    h_td = _gatherh_call(a_td, b_td, etd[0], etd[1], etd[2],
                         ltd1["W1"][:, 2 * d])
    s_bu = _scatter_call(h_bu, ebu[3])
    s_td = _scatter_call(h_td, etd[3])

    pw = params["proj_W"]
    return _post2_proj(
        s_bu, lbu1["W2"].T, x_bu, ebu[4], lbu1["b2"],
        s_td, ltd1["W2"].T, x_td, etd[4], ltd1["b2"],
        pw[:, :d].T, pw[:, d:].T, params["proj_b"],
    )


# speedup vs baseline: 1.1142x; 1.0293x over previous
import functools

import jax
import jax.numpy as jnp
from jax import lax
from jax.experimental import pallas as pl
from jax.experimental.pallas import tpu as pltpu
from jax.experimental.pallas import tpu_sc as plsc

_N = 50000
_D = 256
_E = 49999
_EP = 50176
_TILE_E = _EP // 32
_NGRP = _TILE_E // 16
_NPAD = 50176
_TRASH = 50100


def _gatherh_body(a_hbm, b_hbm, src_hbm, dst_hbm, bl_hbm, w1c_hbm, h_hbm,
                  src_s, dst_s, bl_s, a16a, b16a, a16b, b16b, h16a, h16b,
                  w1c_v, sema, semb, sema2, semb2, semha, semhb):
    c = lax.axis_index("c")
    s = lax.axis_index("s")
    w = s * 2 + c
    ebase = w * _TILE_E

    pltpu.sync_copy(src_hbm.at[pl.ds(ebase, _TILE_E)], src_s)
    pltpu.sync_copy(dst_hbm.at[pl.ds(ebase, _TILE_E)], dst_s)
    pltpu.sync_copy(bl_hbm.at[pl.ds(ebase, _TILE_E)], bl_s)
    pltpu.sync_copy(w1c_hbm, w1c_v)

    def _issue(g, abuf, bbuf, s1, s2):
        idxs = src_s[pl.ds(g * 16, 16)]
        idxg = dst_s[pl.ds(g * 16, 16)]
        cpa = pltpu.async_copy(a_hbm.at[idxs], abuf, s1)
        cpb = pltpu.async_copy(b_hbm.at[idxg], bbuf, s2)
        return cpa, cpb

    wvs = [w1c_v[pl.ds(cc * 16, 16)] for cc in range(16)]

    def _compute(g, abuf, bbuf, hbuf, hsem, it):
        @pl.when(it > 0)
        def _():
            pltpu.make_async_copy(
                hbuf, h_hbm.at[pl.ds(ebase, 16)], hsem
            ).wait()
        for j in range(16):
            blj = plsc.load_gather(
                bl_s, [jnp.full((16,), 0, jnp.int32) + (g * 16 + j)]
            )
            for cc in range(16):
                av = abuf[j, pl.ds(cc * 16, 16)]
                bv = bbuf[j, pl.ds(cc * 16, 16)]
                hbuf[j, pl.ds(cc * 16, 16)] = jnp.maximum(
                    av + bv + blj * wvs[cc], 0.0
                )
        pltpu.async_copy(hbuf, h_hbm.at[pl.ds(ebase + g * 16, 16)], hsem)

    cpa0, cpb0 = _issue(0, a16a, b16a, sema, semb)

    def pair_body(it, _):
        g0 = 2 * it
        pltpu.make_async_copy(a_hbm.at[pl.ds(0, 16)], a16a, sema).wait()
        pltpu.make_async_copy(b_hbm.at[pl.ds(0, 16)], b16a, semb).wait()
        _issue(g0 + 1, a16b, b16b, sema2, semb2)
        _compute(g0, a16a, b16a, h16a, semha, it)
        pltpu.make_async_copy(a_hbm.at[pl.ds(0, 16)], a16b, sema2).wait()
        pltpu.make_async_copy(b_hbm.at[pl.ds(0, 16)], b16b, semb2).wait()

        @pl.when(it + 1 < _NGRP // 2)
        def _():
            _issue(g0 + 2, a16a, b16a, sema, semb)

        _compute(g0 + 1, a16b, b16b, h16b, semhb, it)
        return 0

    lax.fori_loop(0, _NGRP // 2, pair_body, 0)
    pltpu.make_async_copy(h16a, h_hbm.at[pl.ds(ebase, 16)], semha).wait()
    pltpu.make_async_copy(h16b, h_hbm.at[pl.ds(ebase, 16)], semhb).wait()


_gatherh_call = pl.kernel(
    _gatherh_body,
    out_type=jax.ShapeDtypeStruct((_EP, _D), jnp.float32),
    mesh=plsc.VectorSubcoreMesh(
        core_axis_name="c", subcore_axis_name="s", num_cores=2, num_subcores=16
    ),
    compiler_params=pltpu.CompilerParams(needs_layout_passes=False),
    scratch_types=[
        pltpu.VMEM((_TILE_E,), jnp.int32),
        pltpu.VMEM((_TILE_E,), jnp.int32),
        pltpu.VMEM((_TILE_E,), jnp.float32),
        pltpu.VMEM((16, _D), jnp.float32),
        pltpu.VMEM((16, _D), jnp.float32),
        pltpu.VMEM((16, _D), jnp.float32),
        pltpu.VMEM((16, _D), jnp.float32),
        pltpu.VMEM((16, _D), jnp.float32),
        pltpu.VMEM((16, _D), jnp.float32),
        pltpu.VMEM((_D,), jnp.float32),
        pltpu.SemaphoreType.DMA,
        pltpu.SemaphoreType.DMA,
        pltpu.SemaphoreType.DMA,
        pltpu.SemaphoreType.DMA,
        pltpu.SemaphoreType.DMA,
        pltpu.SemaphoreType.DMA,
    ],
)


def _count_body(dst_hbm, z_hbm, cnt_hbm, dst_s, cl):
    c = lax.axis_index("c")
    s = lax.axis_index("s")
    w = s * 2 + c
    ebase = w * _TILE_E

    pltpu.sync_copy(dst_hbm.at[pl.ds(ebase, _TILE_E)], dst_s)
    pltpu.sync_copy(z_hbm, cl)
    lanes = lax.iota(jnp.int32, 16)

    def _lane_take(v, idx):
        return lax.gather(
            v,
            idx[:, None],
            lax.GatherDimensionNumbers(
                offset_dims=(),
                collapsed_slice_dims=(0,),
                start_index_map=(0,),
            ),
            (1,),
            mode=lax.GatherScatterMode.PROMISE_IN_BOUNDS,
        )

    def chunk_body(i, _):
        d16 = dst_s[pl.ds(i * 16, 16)]
        later = jnp.zeros((16,), jnp.int32)
        earlier = jnp.zeros((16,), jnp.int32)
        for sh in range(1, 16):
            up = _lane_take(d16, jnp.minimum(lanes + sh, 15))
            later = later + jnp.where(
                (lanes + sh <= 15) & (d16 == up), 1, 0
            )
            dn = _lane_take(d16, jnp.maximum(lanes - sh, 0))
            earlier = earlier + jnp.where(
                (lanes - sh >= 0) & (d16 == dn), 1, 0
            )
        rep = earlier == 0
        total = (later + 1).astype(jnp.float32)
        plsc.addupdate_scatter(cl, [d16], total, mask=rep)
        return 0

    lax.fori_loop(0, _NGRP, chunk_body, 0)
    pltpu.sync_copy(cl, cnt_hbm.at[pl.ds(w * _NPAD, _NPAD)])


_count_call = pl.kernel(
    _count_body,
    out_type=jax.ShapeDtypeStruct((32 * _NPAD,), jnp.float32),
    mesh=plsc.VectorSubcoreMesh(
        core_axis_name="c", subcore_axis_name="s", num_cores=2, num_subcores=16
    ),
    compiler_params=pltpu.CompilerParams(needs_layout_passes=False),
    scratch_types=[
        pltpu.VMEM((_TILE_E,), jnp.int32),
        pltpu.VMEM((_NPAD,), jnp.float32),
    ],
)


def _scatter_kernel(h_ref, dst_ref, o_ref):
    @pl.when(pl.program_id(0) == 0)
    def _():
        o_ref[...] = jnp.zeros((_NPAD, _D), jnp.float32)

    def eb(j8, _):
        for u in range(16):
            j = j8 * 16 + u
            d = dst_ref[0, 0, j]
            o_ref[pl.ds(d, 1), :] += h_ref[pl.ds(j, 1), :]
        return 0

    lax.fori_loop(0, _TILE_E // 16, eb, 0)


def _scatter_call(h2, dst3d):
    return pl.pallas_call(
        _scatter_kernel,
        grid=(_EP // _TILE_E,),
        in_specs=[
            pl.BlockSpec((_TILE_E, _D), lambda i: (i, 0)),
            pl.BlockSpec((1, 1, _TILE_E), lambda i: (i, 0, 0),
                         memory_space=pltpu.SMEM),
        ],
        out_specs=pl.BlockSpec((_NPAD, _D), lambda i: (0, 0)),
        out_shape=jax.ShapeDtypeStruct((_NPAD, _D), jnp.float32),
        compiler_params=pltpu.CompilerParams(
            vmem_limit_bytes=60 * 1024 * 1024,
        ),
    )(h2, dst3d)


def _reduce_kernel(c_ref, o_ref):
    o_ref[...] = jnp.sum(c_ref[...], axis=0, keepdims=True)


def _reduce_counts(cnt32, bn=6272):
    return pl.pallas_call(
        _reduce_kernel,
        grid=(_NPAD // bn,),
        in_specs=[pl.BlockSpec((32, bn), lambda i: (0, i))],
        out_specs=pl.BlockSpec((1, bn), lambda i: (0, i)),
        out_shape=jax.ShapeDtypeStruct((1, _NPAD), jnp.float32),
    )(cnt32)


def _mm2_kernel(x_ref, wa_ref, wb_ref, ba_ref, oa_ref, ob_ref):
    oa_ref[...] = (
        jnp.dot(x_ref[...], wa_ref[...], preferred_element_type=jnp.float32)
        + ba_ref[...]
    )
    ob_ref[...] = jnp.dot(
        x_ref[...], wb_ref[...], preferred_element_type=jnp.float32
    )


def _matmul_ab(x, wa_t, wb_t, ba, bm=400):
    M, K = x.shape
    N = wa_t.shape[1]
    return pl.pallas_call(
        _mm2_kernel,
        grid=(M // bm,),
        in_specs=[
            pl.BlockSpec((bm, K), lambda i: (i, 0)),
            pl.BlockSpec((K, N), lambda i: (0, 0)),
            pl.BlockSpec((K, N), lambda i: (0, 0)),
            pl.BlockSpec((1, N), lambda i: (0, 0)),
        ],
        out_specs=[
            pl.BlockSpec((bm, N), lambda i: (i, 0)),
            pl.BlockSpec((bm, N), lambda i: (i, 0)),
        ],
        out_shape=[
            jax.ShapeDtypeStruct((M, N), jnp.float32),
            jax.ShapeDtypeStruct((M, N), jnp.float32),
        ],
    )(x, wa_t, wb_t, ba[None, :])


def _post_ab_kernel(s_ref, w2_ref, x_ref, cnt_ref, b2_ref, wa_ref, wb_ref,
                    ba_ref, ox_ref, oa_ref, ob_ref):
    xn = (
        x_ref[...]
        + jnp.dot(s_ref[...], w2_ref[...], preferred_element_type=jnp.float32)
        + cnt_ref[...] * b2_ref[...]
    )
    ox_ref[...] = xn
    oa_ref[...] = (
        jnp.dot(xn, wa_ref[...], preferred_element_type=jnp.float32)
        + ba_ref[...]
    )
    ob_ref[...] = jnp.dot(
        xn, wb_ref[...], preferred_element_type=jnp.float32
    )


def _post_ab(s, w2_t, x, cnt_col, b2, wa_t, wb_t, ba, bm=400):
    M = x.shape[0]
    K = s.shape[1]
    N = w2_t.shape[1]
    return pl.pallas_call(
        _post_ab_kernel,
        grid=(M // bm,),
        in_specs=[
            pl.BlockSpec((bm, K), lambda i: (i, 0)),
            pl.BlockSpec((K, N), lambda i: (0, 0)),
            pl.BlockSpec((bm, N), lambda i: (i, 0)),
            pl.BlockSpec((bm, 1), lambda i: (i, 0)),
            pl.BlockSpec((1, N), lambda i: (0, 0)),
            pl.BlockSpec((N, N), lambda i: (0, 0)),
            pl.BlockSpec((N, N), lambda i: (0, 0)),
            pl.BlockSpec((1, N), lambda i: (0, 0)),
        ],
        out_specs=[
            pl.BlockSpec((bm, N), lambda i: (i, 0)),
            pl.BlockSpec((bm, N), lambda i: (i, 0)),
            pl.BlockSpec((bm, N), lambda i: (i, 0)),
        ],
        out_shape=[
            jax.ShapeDtypeStruct((M, N), jnp.float32),
            jax.ShapeDtypeStruct((M, N), jnp.float32),
            jax.ShapeDtypeStruct((M, N), jnp.float32),
        ],
    )(s, w2_t, x, cnt_col, b2[None, :], wa_t, wb_t, ba[None, :])


def _mm4_kernel(x_ref, wabu_ref, wbbu_ref, babu_ref, watd_ref, wbtd_ref,
                batd_ref, oabu_ref, obbu_ref, oatd_ref, obtd_ref):
    x = x_ref[...]
    oabu_ref[...] = (
        jnp.dot(x, wabu_ref[...], preferred_element_type=jnp.float32)
        + babu_ref[...]
    )
    obbu_ref[...] = jnp.dot(x, wbbu_ref[...], preferred_element_type=jnp.float32)
    oatd_ref[...] = (
        jnp.dot(x, watd_ref[...], preferred_element_type=jnp.float32)
        + batd_ref[...]
    )
    obtd_ref[...] = jnp.dot(x, wbtd_ref[...], preferred_element_type=jnp.float32)


def _matmul_ab4(x, wabu, wbbu, babu, watd, wbtd, batd, bm=400):
    M, K = x.shape
    N = wabu.shape[1]
    row = lambda i: (i, 0)
    full = lambda i: (0, 0)
    return pl.pallas_call(
        _mm4_kernel,
        grid=(M // bm,),
        in_specs=[
            pl.BlockSpec((bm, K), row),
            pl.BlockSpec((K, N), full),
            pl.BlockSpec((K, N), full),
            pl.BlockSpec((1, N), full),
            pl.BlockSpec((K, N), full),
            pl.BlockSpec((K, N), full),
            pl.BlockSpec((1, N), full),
        ],
        out_specs=[pl.BlockSpec((bm, N), row)] * 4,
        out_shape=[jax.ShapeDtypeStruct((M, N), jnp.float32)] * 4,
    )(x, wabu, wbbu, babu[None, :], watd, wbtd, batd[None, :])


def _post_kernel(s_ref, w_ref, x_ref, cnt_ref, b_ref, o_ref):
    o_ref[...] = (
        x_ref[...]
        + jnp.dot(s_ref[...], w_ref[...], preferred_element_type=jnp.float32)
        + cnt_ref[...] * b_ref[...]
    )


def _post_matmul(s, w2_t, x, cnt_col, b2, bm=400):
    M = x.shape[0]
    K = s.shape[1]
    N = w2_t.shape[1]
    return pl.pallas_call(
        _post_kernel,
        grid=(M // bm,),
        in_specs=[
            pl.BlockSpec((bm, K), lambda i: (i, 0)),
            pl.BlockSpec((K, N), lambda i: (0, 0)),
            pl.BlockSpec((bm, N), lambda i: (i, 0)),
            pl.BlockSpec((bm, 1), lambda i: (i, 0)),
            pl.BlockSpec((1, N), lambda i: (0, 0)),
        ],
        out_specs=pl.BlockSpec((bm, N), lambda i: (i, 0)),
        out_shape=jax.ShapeDtypeStruct((M, N), jnp.float32),
    )(s, w2_t, x, cnt_col, b2[None, :])


def _post2_proj_kernel(sbu_ref, w2bu_ref, xbu_ref, cbu_ref, b2bu_ref,
                       std_ref, w2td_ref, xtd_ref, ctd_ref, b2td_ref,
                       pa_ref, pb_ref, pbias_ref, o_ref):
    xbu = (
        xbu_ref[...]
        + jnp.dot(sbu_ref[...], w2bu_ref[...],
                  preferred_element_type=jnp.float32)
        + cbu_ref[...] * b2bu_ref[...]
    )
    xtd = (
        xtd_ref[...]
        + jnp.dot(std_ref[...], w2td_ref[...],
                  preferred_element_type=jnp.float32)
        + ctd_ref[...] * b2td_ref[...]
    )
    o_ref[...] = (
        jnp.dot(xbu, pa_ref[...], preferred_element_type=jnp.float32)
        + jnp.dot(xtd, pb_ref[...], preferred_element_type=jnp.float32)
        + pbias_ref[...]
    )


def _post2_proj(s_bu, w2bu_t, x_bu, cbu, b2bu, s_td, w2td_t, x_td, ctd, b2td,
                pa_t, pb_t, pbias, bm=400):
    M, N = x_bu.shape
    row = lambda i: (i, 0)
    full = lambda i: (0, 0)
    return pl.pallas_call(
        _post2_proj_kernel,
        grid=(M // bm,),
        in_specs=[
            pl.BlockSpec((bm, N), row),
            pl.BlockSpec((N, N), full),
            pl.BlockSpec((bm, N), row),
            pl.BlockSpec((bm, 1), row),
            pl.BlockSpec((1, N), full),
            pl.BlockSpec((bm, N), row),
            pl.BlockSpec((N, N), full),
            pl.BlockSpec((bm, N), row),
            pl.BlockSpec((bm, 1), row),
            pl.BlockSpec((1, N), full),
            pl.BlockSpec((N, N), full),
            pl.BlockSpec((N, N), full),
            pl.BlockSpec((1, N), full),
        ],
        out_specs=pl.BlockSpec((bm, N), row),
        out_shape=jax.ShapeDtypeStruct((M, N), jnp.float32),
    )(s_bu, w2bu_t, x_bu, cbu, b2bu[None, :], s_td, w2td_t, x_td, ctd,
      b2td[None, :], pa_t, pb_t, pbias[None, :])


def _proj_kernel(a_ref, b_ref, wa_ref, wb_ref, bias_ref, o_ref):
    o_ref[...] = (
        jnp.dot(a_ref[...], wa_ref[...], preferred_element_type=jnp.float32)
        + jnp.dot(b_ref[...], wb_ref[...], preferred_element_type=jnp.float32)
        + bias_ref[...]
    )


def _final_proj(x_bu, x_td, wa_t, wb_t, bias, bm=400):
    M, K = x_bu.shape
    N = wa_t.shape[1]
    return pl.pallas_call(
        _proj_kernel,
        grid=(M // bm,),
        in_specs=[
            pl.BlockSpec((bm, K), lambda i: (i, 0)),
            pl.BlockSpec((bm, K), lambda i: (i, 0)),
            pl.BlockSpec((K, N), lambda i: (0, 0)),
            pl.BlockSpec((K, N), lambda i: (0, 0)),
            pl.BlockSpec((1, N), lambda i: (0, 0)),
        ],
        out_specs=pl.BlockSpec((bm, N), lambda i: (i, 0)),
        out_shape=jax.ShapeDtypeStruct((M, N), jnp.float32),
    )(x_bu, x_td, wa_t, wb_t, bias[None, :])


def kernel(edge_index, species_ids, branch_lengths, params):
    d = params["internal_embedding"].shape[0]

    x0 = jnp.take(params["species_embedding"], species_ids, axis=0)

    p2c = edge_index[:, 0::2]
    c2p = edge_index[:, 1::2]
    bl_p2c = branch_lengths[0::2]
    bl_c2p = branch_lengths[1::2]

    pad_i = jnp.zeros((_EP - _E,), jnp.int32)
    pad_t = jnp.full((_EP - _E,), _TRASH, jnp.int32)
    pad_f = jnp.zeros((_EP - _E,), jnp.float32)
    zrow = jnp.zeros((_NPAD,), jnp.float32)

    def prep_dir(ei, bl):
        src = jnp.concatenate([ei[0], pad_i])
        dst = jnp.concatenate([ei[1], pad_t])
        blp = jnp.concatenate([bl, pad_f])
        dst3d = dst.reshape(32, 1, _TILE_E)
        cnt32 = _count_call(dst, zrow).reshape(32, _NPAD)
        cnt_col = _reduce_counts(cnt32).reshape(_NPAD, 1)
        return src, dst, blp, dst3d, cnt_col

    ebu = prep_dir(c2p, bl_c2p)
    etd = prep_dir(p2c, bl_p2c)
    lbu0, lbu1 = params["bu"]
    ltd0, ltd1 = params["td"]
    a_bu, b_bu, a_td, b_td = _matmul_ab4(
        x0,
        lbu0["W1"][:, :d].T, lbu0["W1"][:, d : 2 * d].T, lbu0["b1"],
        ltd0["W1"][:, :d].T, ltd0["W1"][:, d : 2 * d].T, ltd0["b1"],
    )
    h_bu = _gatherh_call(a_bu, b_bu, ebu[0], ebu[1], ebu[2],
                         lbu0["W1"][:, 2 * d])
    h_td = _gatherh_call(a_td, b_td, etd[0], etd[1], etd[2],
                         ltd0["W1"][:, 2 * d])
    s_bu = _scatter_call(h_bu, ebu[3])
    s_td = _scatter_call(h_td, etd[3])
    x_bu, a_bu, b_bu = _post_ab(
        s_bu, lbu0["W2"].T, x0, ebu[4], lbu0["b2"],
        lbu1["W1"][:, :d].T, lbu1["W1"][:, d : 2 * d].T, lbu1["b1"]
    )
    x_td, a_td, b_td = _post_ab(
        s_td, ltd0["W2"].T, x0, etd[4], ltd0["b2"],
        ltd1["W1"][:, :d].T, ltd1["W1"][:, d : 2 * d].T, ltd1["b1"]
    )
    h_bu = _gatherh_call(a_bu, b_bu, ebu[0], ebu[1], ebu[2],
                         lbu1["W1"][:, 2 * d])
    h_td = _gatherh_call(a_td, b_td, etd[0], etd[1], etd[2],
                         ltd1["W1"][:, 2 * d])
    s_bu = _scatter_call(h_bu, ebu[3])
    s_td = _scatter_call(h_td, etd[3])

    pw = params["proj_W"]
    return _post2_proj(
        s_bu, lbu1["W2"].T, x_bu, ebu[4], lbu1["b2"],
        s_td, ltd1["W2"].T, x_td, etd[4], ltd1["b2"],
        pw[:, :d].T, pw[:, d:].T, params["proj_b"],
    )
